# baseline (speedup 1.0000x reference)
import functools

import jax
import jax.numpy as jnp
from jax import lax
from jax.experimental import pallas as pl
from jax.experimental.pallas import tpu as pltpu
from jax.experimental.pallas import tpu_sc as plsc

N = 10000
D = 128
E = 320000
EPS = 1e-5
BN_SCALE = 1.0 / (1.0 + EPS) ** 0.5

NC = 2
NS = 16
NW = NC * NS
CHUNK = 128
CPW = 80
HCPW = CPW // 2
E_PAD = NW * CPW * CHUNK
N_PAD = 10112
RPS = N_PAD // NS
TRASH = N
BLK = 128
NBLK = N_PAD // BLK
HIST_W = 16


def _sc_mesh():
    return plsc.VectorSubcoreMesh(core_axis_name="c", subcore_axis_name="s",
                                  num_cores=NC, num_subcores=NS)


def _sc_deg_body(dst_hbm, ones_hbm, zeros_hbm, out_hbm, didx, ones_c, zbuf,
                 hist):
    c = lax.axis_index("c")
    s = lax.axis_index("s")
    wid = c * NS + s
    r0 = pl.multiple_of(s * RPS, 8)
    pltpu.sync_copy(ones_hbm, ones_c)
    pltpu.sync_copy(zeros_hbm.at[pl.ds(0, RPS)], zbuf)
    pltpu.sync_copy(zbuf, hist.at[pl.ds(r0, RPS)])
    plsc.subcore_barrier()
    base = wid * (CPW * CHUNK)

    def chunk(j, carry):
        off = pl.multiple_of(base + j * CHUNK, CHUNK)
        pltpu.sync_copy(dst_hbm.at[pl.ds(off, CHUNK)], didx.at[0])
        pltpu.sync_copy(ones_c, hist.at[didx.at[0]], add=True)
        return carry

    lax.fori_loop(0, CPW, chunk, 0)
    plsc.subcore_barrier()
    o0 = pl.multiple_of(c * N_PAD + r0, 8)
    pltpu.sync_copy(hist.at[pl.ds(r0, RPS)], zbuf)
    pltpu.sync_copy(zbuf, out_hbm.at[pl.ds(o0, RPS)])


def _sc_deg_call(dst_p, ones_c, zeros_rows):
    return pl.kernel(
        _sc_deg_body,
        out_type=jax.ShapeDtypeStruct((NC * N_PAD,), jnp.float32),
        mesh=_sc_mesh(),
        scratch_types=[
            pltpu.VMEM((1, CHUNK), jnp.int32),
            pltpu.VMEM((CHUNK,), jnp.float32),
            pltpu.VMEM((RPS,), jnp.float32),
            pltpu.VMEM_SHARED((N_PAD,), jnp.float32),
        ],
    )(dst_p, ones_c, zeros_rows)


def _sc_edges_body(y_hbm, src_hbm, dst_hbm, out_hbm, sidx, didx, rows, acc, sem):
    c = lax.axis_index("c")
    s = lax.axis_index("s")
    wid = c * NS + s
    r0 = s * RPS
    pltpu.sync_copy(y_hbm.at[pl.ds(r0, RPS)], acc.at[pl.ds(r0, RPS)])
    plsc.subcore_barrier()
    base = wid * (CPW * CHUNK)

    def chunk(j, carry):
        off = pl.multiple_of(base + j * CHUNK, CHUNK)
        pltpu.sync_copy(src_hbm.at[pl.ds(off, CHUNK)], sidx.at[0])
        pltpu.sync_copy(dst_hbm.at[pl.ds(off, CHUNK)], didx.at[0])
        pltpu.async_copy(y_hbm.at[sidx.at[0]], rows.at[0], sem).wait()
        pltpu.sync_copy(rows.at[0], acc.at[didx.at[0]], add=True)
        return carry

    lax.fori_loop(0, CPW, chunk, 0)
    plsc.subcore_barrier()
    pltpu.sync_copy(acc.at[pl.ds(r0, RPS)], out_hbm.at[c, pl.ds(r0, RPS)])


def _sc_edges_call(y, src_p, dst_p):
    return pl.kernel(
        _sc_edges_body,
        out_type=jax.ShapeDtypeStruct((NC, N_PAD, D), jnp.float32),
        mesh=_sc_mesh(),
        scratch_types=[
            pltpu.VMEM((1, CHUNK), jnp.int32),
            pltpu.VMEM((1, CHUNK), jnp.int32),
            pltpu.VMEM((1, CHUNK, D), jnp.float32),
            pltpu.VMEM_SHARED((N_PAD, D), jnp.float32),
            pltpu.SemaphoreType.DMA,
        ],
    )(y, src_p, dst_p)


def _matT(a, w):
    return lax.dot_general(a, w, (((1,), (1,)), ((), ())),
                           preferred_element_type=jnp.float32)


def _tc_prep_body(x_ref, w_ref, hist_ref, y_ref, dinv_ref):
    deg = 1.0 + jnp.sum(hist_ref[...], axis=0)
    dinv = lax.rsqrt(deg)[:, None]
    y_ref[...] = _matT(x_ref[...], w_ref[...]) * dinv
    dinv_ref[...] = dinv


def _tc_prep(x_p, w1, hist):
    return pl.pallas_call(
        _tc_prep_body,
        grid=(NBLK,),
        in_specs=[
            pl.BlockSpec((BLK, D), lambda i: (i, 0)),
            pl.BlockSpec((D, D), lambda i: (0, 0)),
            pl.BlockSpec((NC, BLK), lambda i: (0, i)),
        ],
        out_specs=[
            pl.BlockSpec((BLK, D), lambda i: (i, 0)),
            pl.BlockSpec((BLK, 1), lambda i: (i, 0)),
        ],
        out_shape=[
            jax.ShapeDtypeStruct((N_PAD, D), jnp.float32),
            jax.ShapeDtypeStruct((N_PAD, 1), jnp.float32),
        ],
    )(x_p, w1, hist)


def _tc_layer_body(p_ref, y_ref, dinv_ref, b_ref, gamma_ref, beta_ref, w_ref,
                   out_ref):
    p = p_ref[...]
    dinv = dinv_ref[...]
    g = dinv * (p[0] + p[1] - y_ref[...])
    h = (g + b_ref[...]) * (gamma_ref[...] * BN_SCALE) + beta_ref[...]
    h = jnp.maximum(h, 0.0)
    out_ref[...] = _matT(h, w_ref[...]) * dinv


def _tc_layer(p, y, dinv, b, gamma, beta, w_next):
    return pl.pallas_call(
        _tc_layer_body,
        grid=(NBLK,),
        in_specs=[
            pl.BlockSpec((NC, BLK, D), lambda i: (0, i, 0)),
            pl.BlockSpec((BLK, D), lambda i: (i, 0)),
            pl.BlockSpec((BLK, 1), lambda i: (i, 0)),
            pl.BlockSpec((1, D), lambda i: (0, 0)),
            pl.BlockSpec((1, D), lambda i: (0, 0)),
            pl.BlockSpec((1, D), lambda i: (0, 0)),
            pl.BlockSpec((D, D), lambda i: (0, 0)),
        ],
        out_specs=pl.BlockSpec((BLK, D), lambda i: (i, 0)),
        out_shape=jax.ShapeDtypeStruct((N_PAD, D), jnp.float32),
    )(p, y, dinv, b, gamma, beta, w_next)


def _tc_final_body(p_ref, y_ref, dinv_ref, b_ref, gamma_ref, beta_ref,
                   pw1_ref, pb1_ref, pw2_ref, pb2_ref, ne_ref, ge_ref,
                   acc_ref):
    i = pl.program_id(0)

    @pl.when(i == 0)
    def _():
        acc_ref[...] = jnp.zeros_like(acc_ref)

    p = p_ref[...]
    g = dinv_ref[...] * (p[0] + p[1] - y_ref[...])
    emb = (g + b_ref[...]) * (gamma_ref[...] * BN_SCALE) + beta_ref[...]
    ne_ref[...] = emb
    row = i * BLK + lax.broadcasted_iota(jnp.int32, (BLK, 1), 0)
    masked = jnp.where(row < N, emb, 0.0)
    acc_ref[...] += jnp.sum(masked, axis=0, keepdims=True)

    @pl.when(i == NBLK - 1)
    def _():
        gm = acc_ref[...] * (1.0 / N)
        g1 = jnp.maximum(_matT(gm, pw1_ref[...]) + pb1_ref[...], 0.0)
        ge_ref[...] = _matT(g1, pw2_ref[...]) + pb2_ref[...]


def _tc_final(p, y, dinv, b, gamma, beta, pw1, pb1, pw2, pb2):
    return pl.pallas_call(
        _tc_final_body,
        grid=(NBLK,),
        in_specs=[
            pl.BlockSpec((NC, BLK, D), lambda i: (0, i, 0)),
            pl.BlockSpec((BLK, D), lambda i: (i, 0)),
            pl.BlockSpec((BLK, 1), lambda i: (i, 0)),
            pl.BlockSpec((1, D), lambda i: (0, 0)),
            pl.BlockSpec((1, D), lambda i: (0, 0)),
            pl.BlockSpec((1, D), lambda i: (0, 0)),
            pl.BlockSpec((D, D), lambda i: (0, 0)),
            pl.BlockSpec((1, D), lambda i: (0, 0)),
            pl.BlockSpec((D, D), lambda i: (0, 0)),
            pl.BlockSpec((1, D), lambda i: (0, 0)),
        ],
        out_specs=[
            pl.BlockSpec((BLK, D), lambda i: (i, 0)),
            pl.BlockSpec((1, D), lambda i: (0, 0)),
        ],
        out_shape=[
            jax.ShapeDtypeStruct((N_PAD, D), jnp.float32),
            jax.ShapeDtypeStruct((1, D), jnp.float32),
        ],
        scratch_shapes=[pltpu.VMEM((1, D), jnp.float32)],
    )(p, y, dinv, b, gamma, beta, pw1, pb1, pw2, pb2)


def kernel(x, edge_index, W1, b1, gamma1, beta1, W2, b2, gamma2, beta2,
           W3, b3, gamma3, beta3, PW1, Pb1, PW2, Pb2):
    src = edge_index[0].astype(jnp.int32)
    dst = edge_index[1].astype(jnp.int32)
    pad = jnp.full((E_PAD - E,), TRASH, dtype=jnp.int32)
    src_p = jnp.concatenate([src, pad])
    dst_p = jnp.concatenate([dst, pad])
    x_p = jnp.concatenate(
        [x, jnp.zeros((N_PAD - N, D), dtype=jnp.float32)], axis=0)
    ones_c = jnp.ones((CHUNK,), dtype=jnp.float32)
    zeros_rows = jnp.zeros((N_PAD,), dtype=jnp.float32)

    b1r, g1r, be1 = b1[None, :], gamma1[None, :], beta1[None, :]
    b2r, g2r, be2 = b2[None, :], gamma2[None, :], beta2[None, :]
    b3r, g3r, be3 = b3[None, :], gamma3[None, :], beta3[None, :]

    hist = _sc_deg_call(dst_p, ones_c, zeros_rows).reshape(NC, N_PAD)
    y1, dinv = _tc_prep(x_p, W1, hist)
    p1_ = _sc_edges_call(y1, src_p, dst_p)
    y2 = _tc_layer(p1_, y1, dinv, b1r, g1r, be1, W2)
    p2_ = _sc_edges_call(y2, src_p, dst_p)
    y3 = _tc_layer(p2_, y2, dinv, b2r, g2r, be2, W3)
    p3_ = _sc_edges_call(y3, src_p, dst_p)
    ne_pad, ge = _tc_final(p3_, y3, dinv, b3r, g3r, be3, PW1, Pb1[None, :],
                           PW2, Pb2[None, :])
    return ne_pad[:N], ge


# speedup vs baseline: 1.1317x; 1.1317x over previous
import functools

import jax
import jax.numpy as jnp
from jax import lax
from jax.experimental import pallas as pl
from jax.experimental.pallas import tpu as pltpu
from jax.experimental.pallas import tpu_sc as plsc

N = 10000
D = 128
E = 320000
EPS = 1e-5
BN_SCALE = 1.0 / (1.0 + EPS) ** 0.5

NC = 2
NS = 16
NW = NC * NS
CHUNK = 128
CPW = 80
HCPW = CPW // 2
E_PAD = NW * CPW * CHUNK
N_PAD = 10112
RPS = N_PAD // NS
TRASH = N
BLK = 128
NBLK = N_PAD // BLK
HIST_W = 16


def _sc_mesh():
    return plsc.VectorSubcoreMesh(core_axis_name="c", subcore_axis_name="s",
                                  num_cores=NC, num_subcores=NS)


def _sc_deg_body(dst_hbm, ones_hbm, zeros_hbm, out_hbm, didx, ones_c, zbuf,
                 hist):
    c = lax.axis_index("c")
    s = lax.axis_index("s")
    wid = c * NS + s
    r0 = pl.multiple_of(s * RPS, 8)
    pltpu.sync_copy(ones_hbm, ones_c)
    pltpu.sync_copy(zeros_hbm.at[pl.ds(0, RPS)], zbuf)
    pltpu.sync_copy(zbuf, hist.at[pl.ds(r0, RPS)])
    plsc.subcore_barrier()
    base = wid * (CPW * CHUNK)

    def chunk(j, carry):
        off = pl.multiple_of(base + j * CHUNK, CHUNK)
        pltpu.sync_copy(dst_hbm.at[pl.ds(off, CHUNK)], didx.at[0])
        pltpu.sync_copy(ones_c, hist.at[didx.at[0]], add=True)
        return carry

    lax.fori_loop(0, CPW, chunk, 0)
    plsc.subcore_barrier()
    o0 = pl.multiple_of(c * N_PAD + r0, 8)
    pltpu.sync_copy(hist.at[pl.ds(r0, RPS)], zbuf)
    pltpu.sync_copy(zbuf, out_hbm.at[pl.ds(o0, RPS)])


def _sc_deg_call(dst_p, ones_c, zeros_rows):
    return pl.kernel(
        _sc_deg_body,
        out_type=jax.ShapeDtypeStruct((NC * N_PAD,), jnp.float32),
        mesh=_sc_mesh(),
        scratch_types=[
            pltpu.VMEM((1, CHUNK), jnp.int32),
            pltpu.VMEM((CHUNK,), jnp.float32),
            pltpu.VMEM((RPS,), jnp.float32),
            pltpu.VMEM_SHARED((N_PAD,), jnp.float32),
        ],
    )(dst_p, ones_c, zeros_rows)


def _sc_edges_body(y_hbm, src_hbm, dst_hbm, out_hbm, sidx, didx, rows, acc,
                   semg0, semg1):
    c = lax.axis_index("c")
    s = lax.axis_index("s")
    wid = c * NS + s
    r0 = s * RPS
    semg = (semg0, semg1)
    pltpu.sync_copy(y_hbm.at[pl.ds(r0, RPS)], acc.at[pl.ds(r0, RPS)])
    plsc.subcore_barrier()

    for h in (0, 1):
        base = wid * CPW + h * HCPW
        pltpu.sync_copy(src_hbm.at[pl.ds(base, HCPW)], sidx)
        pltpu.sync_copy(dst_hbm.at[pl.ds(base, HCPW)], didx)
        pltpu.async_copy(y_hbm.at[sidx.at[0]], rows.at[0], semg0)
        for j in range(HCPW):
            b = j % 2
            if j + 1 < HCPW:
                pltpu.async_copy(y_hbm.at[sidx.at[j + 1]], rows.at[1 - b],
                                 semg[1 - b])
            pltpu.make_async_copy(y_hbm.at[sidx.at[j]], rows.at[b],
                                  semg[b]).wait()
            pltpu.sync_copy(rows.at[b], acc.at[didx.at[j]], add=True)
    plsc.subcore_barrier()
    pltpu.sync_copy(acc.at[pl.ds(r0, RPS)], out_hbm.at[c, pl.ds(r0, RPS)])


def _sc_edges_call(y, src_p, dst_p):
    return pl.kernel(
        _sc_edges_body,
        out_type=jax.ShapeDtypeStruct((NC, N_PAD, D), jnp.float32),
        mesh=_sc_mesh(),
        scratch_types=[
            pltpu.VMEM((HCPW, CHUNK), jnp.int32),
            pltpu.VMEM((HCPW, CHUNK), jnp.int32),
            pltpu.VMEM((2, CHUNK, D), jnp.float32),
            pltpu.VMEM_SHARED((N_PAD, D), jnp.float32),
            pltpu.SemaphoreType.DMA,
            pltpu.SemaphoreType.DMA,
        ],
    )(y, src_p, dst_p)


def _matT(a, w):
    return lax.dot_general(a, w, (((1,), (1,)), ((), ())),
                           preferred_element_type=jnp.float32)


def _tc_prep_body(x_ref, w_ref, hist_ref, y_ref, dinv_ref):
    deg = 1.0 + jnp.sum(hist_ref[...], axis=0)
    dinv = lax.rsqrt(deg)[:, None]
    y_ref[...] = _matT(x_ref[...], w_ref[...]) * dinv
    dinv_ref[...] = dinv


def _tc_prep(x_p, w1, hist):
    return pl.pallas_call(
        _tc_prep_body,
        grid=(NBLK,),
        in_specs=[
            pl.BlockSpec((BLK, D), lambda i: (i, 0)),
            pl.BlockSpec((D, D), lambda i: (0, 0)),
            pl.BlockSpec((NC, BLK), lambda i: (0, i)),
        ],
        out_specs=[
            pl.BlockSpec((BLK, D), lambda i: (i, 0)),
            pl.BlockSpec((BLK, 1), lambda i: (i, 0)),
        ],
        out_shape=[
            jax.ShapeDtypeStruct((N_PAD, D), jnp.float32),
            jax.ShapeDtypeStruct((N_PAD, 1), jnp.float32),
        ],
    )(x_p, w1, hist)


def _tc_layer_body(p_ref, y_ref, dinv_ref, b_ref, gamma_ref, beta_ref, w_ref,
                   out_ref):
    p = p_ref[...]
    dinv = dinv_ref[...]
    g = dinv * (p[0] + p[1] - y_ref[...])
    h = (g + b_ref[...]) * (gamma_ref[...] * BN_SCALE) + beta_ref[...]
    h = jnp.maximum(h, 0.0)
    out_ref[...] = _matT(h, w_ref[...]) * dinv


def _tc_layer(p, y, dinv, b, gamma, beta, w_next):
    return pl.pallas_call(
        _tc_layer_body,
        grid=(NBLK,),
        in_specs=[
            pl.BlockSpec((NC, BLK, D), lambda i: (0, i, 0)),
            pl.BlockSpec((BLK, D), lambda i: (i, 0)),
            pl.BlockSpec((BLK, 1), lambda i: (i, 0)),
            pl.BlockSpec((1, D), lambda i: (0, 0)),
            pl.BlockSpec((1, D), lambda i: (0, 0)),
            pl.BlockSpec((1, D), lambda i: (0, 0)),
            pl.BlockSpec((D, D), lambda i: (0, 0)),
        ],
        out_specs=pl.BlockSpec((BLK, D), lambda i: (i, 0)),
        out_shape=jax.ShapeDtypeStruct((N_PAD, D), jnp.float32),
    )(p, y, dinv, b, gamma, beta, w_next)


def _tc_final_body(p_ref, y_ref, dinv_ref, b_ref, gamma_ref, beta_ref,
                   pw1_ref, pb1_ref, pw2_ref, pb2_ref, ne_ref, ge_ref,
                   acc_ref):
    i = pl.program_id(0)

    @pl.when(i == 0)
    def _():
        acc_ref[...] = jnp.zeros_like(acc_ref)

    p = p_ref[...]
    g = dinv_ref[...] * (p[0] + p[1] - y_ref[...])
    emb = (g + b_ref[...]) * (gamma_ref[...] * BN_SCALE) + beta_ref[...]
    ne_ref[...] = emb
    row = i * BLK + lax.broadcasted_iota(jnp.int32, (BLK, 1), 0)
    masked = jnp.where(row < N, emb, 0.0)
    acc_ref[...] += jnp.sum(masked, axis=0, keepdims=True)

    @pl.when(i == NBLK - 1)
    def _():
        gm = acc_ref[...] * (1.0 / N)
        g1 = jnp.maximum(_matT(gm, pw1_ref[...]) + pb1_ref[...], 0.0)
        ge_ref[...] = _matT(g1, pw2_ref[...]) + pb2_ref[...]


def _tc_final(p, y, dinv, b, gamma, beta, pw1, pb1, pw2, pb2):
    return pl.pallas_call(
        _tc_final_body,
        grid=(NBLK,),
        in_specs=[
            pl.BlockSpec((NC, BLK, D), lambda i: (0, i, 0)),
            pl.BlockSpec((BLK, D), lambda i: (i, 0)),
            pl.BlockSpec((BLK, 1), lambda i: (i, 0)),
            pl.BlockSpec((1, D), lambda i: (0, 0)),
            pl.BlockSpec((1, D), lambda i: (0, 0)),
            pl.BlockSpec((1, D), lambda i: (0, 0)),
            pl.BlockSpec((D, D), lambda i: (0, 0)),
            pl.BlockSpec((1, D), lambda i: (0, 0)),
            pl.BlockSpec((D, D), lambda i: (0, 0)),
            pl.BlockSpec((1, D), lambda i: (0, 0)),
        ],
        out_specs=[
            pl.BlockSpec((BLK, D), lambda i: (i, 0)),
            pl.BlockSpec((1, D), lambda i: (0, 0)),
        ],
        out_shape=[
            jax.ShapeDtypeStruct((N_PAD, D), jnp.float32),
            jax.ShapeDtypeStruct((1, D), jnp.float32),
        ],
        scratch_shapes=[pltpu.VMEM((1, D), jnp.float32)],
    )(p, y, dinv, b, gamma, beta, pw1, pb1, pw2, pb2)


def kernel(x, edge_index, W1, b1, gamma1, beta1, W2, b2, gamma2, beta2,
           W3, b3, gamma3, beta3, PW1, Pb1, PW2, Pb2):
    src = edge_index[0].astype(jnp.int32)
    dst = edge_index[1].astype(jnp.int32)
    pad = jnp.full((E_PAD - E,), TRASH, dtype=jnp.int32)
    src_flat = jnp.concatenate([src, pad])
    dst_flat = jnp.concatenate([dst, pad])
    src_p = src_flat.reshape(NW * CPW, CHUNK)
    dst_p = dst_flat.reshape(NW * CPW, CHUNK)
    x_p = jnp.concatenate(
        [x, jnp.zeros((N_PAD - N, D), dtype=jnp.float32)], axis=0)
    ones_c = jnp.ones((CHUNK,), dtype=jnp.float32)
    zeros_rows = jnp.zeros((N_PAD,), dtype=jnp.float32)

    b1r, g1r, be1 = b1[None, :], gamma1[None, :], beta1[None, :]
    b2r, g2r, be2 = b2[None, :], gamma2[None, :], beta2[None, :]
    b3r, g3r, be3 = b3[None, :], gamma3[None, :], beta3[None, :]

    hist = _sc_deg_call(dst_flat, ones_c, zeros_rows).reshape(NC, N_PAD)
    y1, dinv = _tc_prep(x_p, W1, hist)
    p1_ = _sc_edges_call(y1, src_p, dst_p)
    y2 = _tc_layer(p1_, y1, dinv, b1r, g1r, be1, W2)
    p2_ = _sc_edges_call(y2, src_p, dst_p)
    y3 = _tc_layer(p2_, y2, dinv, b2r, g2r, be2, W3)
    p3_ = _sc_edges_call(y3, src_p, dst_p)
    ne_pad, ge = _tc_final(p3_, y3, dinv, b3r, g3r, be3, PW1, Pb1[None, :],
                           PW2, Pb2[None, :])
    return ne_pad[:N], ge


# speedup vs baseline: 2.1760x; 1.9228x over previous
import functools

import jax
import jax.numpy as jnp
from jax import lax
from jax.experimental import pallas as pl
from jax.experimental.pallas import tpu as pltpu
from jax.experimental.pallas import tpu_sc as plsc

N = 10000
D = 128
E = 320000
EPS = 1e-5
BN_SCALE = 1.0 / (1.0 + EPS) ** 0.5

NC = 2
NS = 16
NW = NC * NS
CHUNK = 120
CPW = 84
E_PAD = NW * CPW * CHUNK
N_PAD = 10112
RPS = N_PAD // NS
TRASH = N
BLK = 128
NBLK = N_PAD // BLK
HIST_W = 16


def _sc_mesh():
    return plsc.VectorSubcoreMesh(core_axis_name="c", subcore_axis_name="s",
                                  num_cores=NC, num_subcores=NS)


def _sc_deg_body(dst_hbm, ones_hbm, zeros_hbm, out_hbm, didx, ones_c, zbuf,
                 hist):
    c = lax.axis_index("c")
    s = lax.axis_index("s")
    wid = c * NS + s
    r0 = pl.multiple_of(s * RPS, 8)
    pltpu.sync_copy(ones_hbm, ones_c)
    pltpu.sync_copy(zeros_hbm.at[pl.ds(0, RPS)], zbuf)
    pltpu.sync_copy(zbuf, hist.at[pl.ds(r0, RPS)])
    plsc.subcore_barrier()
    base = wid * (CPW * CHUNK)

    def chunk(j, carry):
        off = pl.multiple_of(base + j * CHUNK, CHUNK)
        pltpu.sync_copy(dst_hbm.at[pl.ds(off, CHUNK)], didx.at[0])
        pltpu.sync_copy(ones_c, hist.at[didx.at[0]], add=True)
        return carry

    lax.fori_loop(0, CPW, chunk, 0)
    plsc.subcore_barrier()
    o0 = pl.multiple_of(c * N_PAD + r0, 8)
    pltpu.sync_copy(hist.at[pl.ds(r0, RPS)], zbuf)
    pltpu.sync_copy(zbuf, out_hbm.at[pl.ds(o0, RPS)])


def _sc_deg_call(dst_p, ones_c, zeros_rows):
    return pl.kernel(
        _sc_deg_body,
        out_type=jax.ShapeDtypeStruct((NC * N_PAD,), jnp.float32),
        mesh=_sc_mesh(),
        scratch_types=[
            pltpu.VMEM((1, CHUNK), jnp.int32),
            pltpu.VMEM((CHUNK,), jnp.float32),
            pltpu.VMEM((RPS,), jnp.float32),
            pltpu.VMEM_SHARED((N_PAD,), jnp.float32),
        ],
    )(dst_p, ones_c, zeros_rows)


def _sc_edges_body(y_hbm, src_hbm, dst_hbm, out_hbm,
                   si0, si1, si2, si3, di0, di1, di2, di3,
                   ra, rb, rc, acc,
                   sg0, sg1, sg2, ss0, ss1, ss2, sl0, sl1, sl2, sl3):
    c = lax.axis_index("c")
    s = lax.axis_index("s")
    wid = c * NS + s
    r0 = s * RPS
    sidx = (si0, si1, si2, si3)
    didx = (di0, di1, di2, di3)
    rows = (ra, rb, rc)
    semg = (sg0, sg1, sg2)
    sems = (ss0, ss1, ss2)
    semi = (sl0, sl1, sl2, sl3)
    base = wid * (CPW * CHUNK)
    pltpu.sync_copy(y_hbm.at[pl.ds(r0, RPS)], acc.at[pl.ds(r0, RPS)])
    for t in range(3):
        off = base + t * CHUNK
        pltpu.sync_copy(src_hbm.at[pl.ds(off, CHUNK)], sidx[t].at[0])
        pltpu.sync_copy(dst_hbm.at[pl.ds(off, CHUNK)], didx[t].at[0])
    plsc.subcore_barrier()

    pltpu.async_copy(y_hbm.at[sidx[0].at[0]], rows[0], semg[0])
    pltpu.async_copy(y_hbm.at[sidx[1].at[0]], rows[1], semg[1])
    for j in range(CPW):
        b = j % 3
        q = j % 4
        if j + 2 < CPW:
            if j >= 1:
                pltpu.make_async_copy(src_hbm.at[pl.ds(base, CHUNK)],
                                      sidx[(j + 2) % 4].at[0],
                                      semi[(j + 2) % 4]).wait()
                pltpu.make_async_copy(dst_hbm.at[pl.ds(base, CHUNK)],
                                      didx[(j + 2) % 4].at[0],
                                      semi[(j + 2) % 4]).wait()
                pltpu.make_async_copy(rows[(j - 1) % 3],
                                      acc.at[didx[(j - 1) % 4].at[0]],
                                      sems[(j - 1) % 3]).wait()
            pltpu.async_copy(y_hbm.at[sidx[(j + 2) % 4].at[0]],
                             rows[(j + 2) % 3], semg[(j + 2) % 3])
        pltpu.make_async_copy(y_hbm.at[sidx[q].at[0]], rows[b],
                              semg[b]).wait()
        pltpu.async_copy(rows[b], acc.at[didx[q].at[0]], sems[b], add=True)
        if j + 3 < CPW:
            off = base + (j + 3) * CHUNK
            pltpu.async_copy(src_hbm.at[pl.ds(off, CHUNK)],
                             sidx[(j + 3) % 4].at[0], semi[(j + 3) % 4])
            pltpu.async_copy(dst_hbm.at[pl.ds(off, CHUNK)],
                             didx[(j + 3) % 4].at[0], semi[(j + 3) % 4])
    for k in (CPW - 3, CPW - 2, CPW - 1):
        pltpu.make_async_copy(rows[k % 3], acc.at[didx[k % 4].at[0]],
                              sems[k % 3]).wait()
    plsc.subcore_barrier()
    pltpu.sync_copy(acc.at[pl.ds(r0, RPS)], out_hbm.at[c, pl.ds(r0, RPS)])


def _sc_edges_call(y, src_p, dst_p):
    idx = pltpu.VMEM((1, CHUNK), jnp.int32)
    rbuf = pltpu.VMEM((CHUNK, D), jnp.float32)
    sem = pltpu.SemaphoreType.DMA
    return pl.kernel(
        _sc_edges_body,
        out_type=jax.ShapeDtypeStruct((NC, N_PAD, D), jnp.float32),
        mesh=_sc_mesh(),
        scratch_types=[idx] * 8 + [rbuf] * 3 + [
            pltpu.VMEM_SHARED((N_PAD, D), jnp.float32)] + [sem] * 10,
    )(y, src_p, dst_p)


def _matT(a, w):
    return lax.dot_general(a, w, (((1,), (1,)), ((), ())),
                           preferred_element_type=jnp.float32)


def _tc_prep_body(x_ref, w_ref, hist_ref, y_ref, dinv_ref):
    deg = 1.0 + jnp.sum(hist_ref[...], axis=0)
    dinv = lax.rsqrt(deg)[:, None]
    y_ref[...] = _matT(x_ref[...], w_ref[...]) * dinv
    dinv_ref[...] = dinv


def _tc_prep(x_p, w1, hist):
    return pl.pallas_call(
        _tc_prep_body,
        grid=(NBLK,),
        in_specs=[
            pl.BlockSpec((BLK, D), lambda i: (i, 0)),
            pl.BlockSpec((D, D), lambda i: (0, 0)),
            pl.BlockSpec((NC, BLK), lambda i: (0, i)),
        ],
        out_specs=[
            pl.BlockSpec((BLK, D), lambda i: (i, 0)),
            pl.BlockSpec((BLK, 1), lambda i: (i, 0)),
        ],
        out_shape=[
            jax.ShapeDtypeStruct((N_PAD, D), jnp.float32),
            jax.ShapeDtypeStruct((N_PAD, 1), jnp.float32),
        ],
    )(x_p, w1, hist)


def _tc_layer_body(p_ref, y_ref, dinv_ref, b_ref, gamma_ref, beta_ref, w_ref,
                   out_ref):
    p = p_ref[...]
    dinv = dinv_ref[...]
    g = dinv * (p[0] + p[1] - y_ref[...])
    h = (g + b_ref[...]) * (gamma_ref[...] * BN_SCALE) + beta_ref[...]
    h = jnp.maximum(h, 0.0)
    out_ref[...] = _matT(h, w_ref[...]) * dinv


def _tc_layer(p, y, dinv, b, gamma, beta, w_next):
    return pl.pallas_call(
        _tc_layer_body,
        grid=(NBLK,),
        in_specs=[
            pl.BlockSpec((NC, BLK, D), lambda i: (0, i, 0)),
            pl.BlockSpec((BLK, D), lambda i: (i, 0)),
            pl.BlockSpec((BLK, 1), lambda i: (i, 0)),
            pl.BlockSpec((1, D), lambda i: (0, 0)),
            pl.BlockSpec((1, D), lambda i: (0, 0)),
            pl.BlockSpec((1, D), lambda i: (0, 0)),
            pl.BlockSpec((D, D), lambda i: (0, 0)),
        ],
        out_specs=pl.BlockSpec((BLK, D), lambda i: (i, 0)),
        out_shape=jax.ShapeDtypeStruct((N_PAD, D), jnp.float32),
    )(p, y, dinv, b, gamma, beta, w_next)


def _tc_final_body(p_ref, y_ref, dinv_ref, b_ref, gamma_ref, beta_ref,
                   pw1_ref, pb1_ref, pw2_ref, pb2_ref, ne_ref, ge_ref,
                   acc_ref):
    i = pl.program_id(0)

    @pl.when(i == 0)
    def _():
        acc_ref[...] = jnp.zeros_like(acc_ref)

    p = p_ref[...]
    g = dinv_ref[...] * (p[0] + p[1] - y_ref[...])
    emb = (g + b_ref[...]) * (gamma_ref[...] * BN_SCALE) + beta_ref[...]
    ne_ref[...] = emb
    row = i * BLK + lax.broadcasted_iota(jnp.int32, (BLK, 1), 0)
    masked = jnp.where(row < N, emb, 0.0)
    acc_ref[...] += jnp.sum(masked, axis=0, keepdims=True)

    @pl.when(i == NBLK - 1)
    def _():
        gm = acc_ref[...] * (1.0 / N)
        g1 = jnp.maximum(_matT(gm, pw1_ref[...]) + pb1_ref[...], 0.0)
        ge_ref[...] = _matT(g1, pw2_ref[...]) + pb2_ref[...]


def _tc_final(p, y, dinv, b, gamma, beta, pw1, pb1, pw2, pb2):
    return pl.pallas_call(
        _tc_final_body,
        grid=(NBLK,),
        in_specs=[
            pl.BlockSpec((NC, BLK, D), lambda i: (0, i, 0)),
            pl.BlockSpec((BLK, D), lambda i: (i, 0)),
            pl.BlockSpec((BLK, 1), lambda i: (i, 0)),
            pl.BlockSpec((1, D), lambda i: (0, 0)),
            pl.BlockSpec((1, D), lambda i: (0, 0)),
            pl.BlockSpec((1, D), lambda i: (0, 0)),
            pl.BlockSpec((D, D), lambda i: (0, 0)),
            pl.BlockSpec((1, D), lambda i: (0, 0)),
            pl.BlockSpec((D, D), lambda i: (0, 0)),
            pl.BlockSpec((1, D), lambda i: (0, 0)),
        ],
        out_specs=[
            pl.BlockSpec((BLK, D), lambda i: (i, 0)),
            pl.BlockSpec((1, D), lambda i: (0, 0)),
        ],
        out_shape=[
            jax.ShapeDtypeStruct((N_PAD, D), jnp.float32),
            jax.ShapeDtypeStruct((1, D), jnp.float32),
        ],
        scratch_shapes=[pltpu.VMEM((1, D), jnp.float32)],
    )(p, y, dinv, b, gamma, beta, pw1, pb1, pw2, pb2)


def kernel(x, edge_index, W1, b1, gamma1, beta1, W2, b2, gamma2, beta2,
           W3, b3, gamma3, beta3, PW1, Pb1, PW2, Pb2):
    src = edge_index[0].astype(jnp.int32)
    dst = edge_index[1].astype(jnp.int32)
    pad = jnp.full((E_PAD - E,), TRASH, dtype=jnp.int32)
    src_p = jnp.concatenate([src, pad])
    dst_p = jnp.concatenate([dst, pad])
    x_p = jnp.concatenate(
        [x, jnp.zeros((N_PAD - N, D), dtype=jnp.float32)], axis=0)
    ones_c = jnp.ones((CHUNK,), dtype=jnp.float32)
    zeros_rows = jnp.zeros((N_PAD,), dtype=jnp.float32)

    b1r, g1r, be1 = b1[None, :], gamma1[None, :], beta1[None, :]
    b2r, g2r, be2 = b2[None, :], gamma2[None, :], beta2[None, :]
    b3r, g3r, be3 = b3[None, :], gamma3[None, :], beta3[None, :]

    hist = _sc_deg_call(dst_p, ones_c, zeros_rows).reshape(NC, N_PAD)
    y1, dinv = _tc_prep(x_p, W1, hist)
    p1_ = _sc_edges_call(y1, src_p, dst_p)
    y2 = _tc_layer(p1_, y1, dinv, b1r, g1r, be1, W2)
    p2_ = _sc_edges_call(y2, src_p, dst_p)
    y3 = _tc_layer(p2_, y2, dinv, b2r, g2r, be2, W3)
    p3_ = _sc_edges_call(y3, src_p, dst_p)
    ne_pad, ge = _tc_final(p3_, y3, dinv, b3r, g3r, be3, PW1, Pb1[None, :],
                           PW2, Pb2[None, :])
    return ne_pad[:N], ge


# speedup vs baseline: 2.7121x; 1.2463x over previous
import functools

import jax
import jax.numpy as jnp
from jax import lax
from jax.experimental import pallas as pl
from jax.experimental.pallas import tpu as pltpu
from jax.experimental.pallas import tpu_sc as plsc

N = 10000
D = 128
E = 320000
EPS = 1e-5
BN_SCALE = 1.0 / (1.0 + EPS) ** 0.5

NC = 2
NS = 16
NW = NC * NS
CHUNK = 88
CPW = 114
E_PAD = NW * CPW * CHUNK
N_PAD = 10112
RPS = N_PAD // NS
TRASH = N
BLK = 128
NBLK = N_PAD // BLK
HIST_W = 16


def _sc_mesh():
    return plsc.VectorSubcoreMesh(core_axis_name="c", subcore_axis_name="s",
                                  num_cores=NC, num_subcores=NS)


def _sc_deg_body(dst_hbm, ones_hbm, zeros_hbm, out_hbm, didx, ones_c, zbuf,
                 hist):
    c = lax.axis_index("c")
    s = lax.axis_index("s")
    wid = c * NS + s
    r0 = pl.multiple_of(s * RPS, 8)
    pltpu.sync_copy(ones_hbm, ones_c)
    pltpu.sync_copy(zeros_hbm.at[pl.ds(0, RPS)], zbuf)
    pltpu.sync_copy(zbuf, hist.at[pl.ds(r0, RPS)])
    plsc.subcore_barrier()
    base = wid * (CPW * CHUNK)

    def chunk(j, carry):
        off = pl.multiple_of(base + j * CHUNK, CHUNK)
        pltpu.sync_copy(dst_hbm.at[pl.ds(off, CHUNK)], didx.at[0])
        pltpu.sync_copy(ones_c, hist.at[didx.at[0]], add=True)
        return carry

    lax.fori_loop(0, CPW, chunk, 0)
    plsc.subcore_barrier()
    o0 = pl.multiple_of(c * N_PAD + r0, 8)
    pltpu.sync_copy(hist.at[pl.ds(r0, RPS)], zbuf)
    pltpu.sync_copy(zbuf, out_hbm.at[pl.ds(o0, RPS)])


def _sc_deg_call(dst_p, ones_c, zeros_rows):
    return pl.kernel(
        _sc_deg_body,
        out_type=jax.ShapeDtypeStruct((NC * N_PAD,), jnp.float32),
        mesh=_sc_mesh(),
        scratch_types=[
            pltpu.VMEM((1, CHUNK), jnp.int32),
            pltpu.VMEM((CHUNK,), jnp.float32),
            pltpu.VMEM((RPS,), jnp.float32),
            pltpu.VMEM_SHARED((N_PAD,), jnp.float32),
        ],
    )(dst_p, ones_c, zeros_rows)


def _sc_edges_body(y_hbm, src_hbm, dst_hbm, out_hbm,
                   si0, si1, si2, si3, si4, di0, di1, di2, di3, di4,
                   ra, rb, rc, rd, acc,
                   sg0, sg1, sg2, sg3, ss0, ss1, ss2, ss3,
                   sl0, sl1, sl2, sl3, sl4):
    c = lax.axis_index("c")
    s = lax.axis_index("s")
    wid = c * NS + s
    r0 = s * RPS
    sidx = (si0, si1, si2, si3, si4)
    didx = (di0, di1, di2, di3, di4)
    rows = (ra, rb, rc, rd)
    semg = (sg0, sg1, sg2, sg3)
    sems = (ss0, ss1, ss2, ss3)
    semi = (sl0, sl1, sl2, sl3, sl4)
    base = wid * (CPW * CHUNK)
    pltpu.sync_copy(y_hbm.at[pl.ds(r0, RPS)], acc.at[pl.ds(r0, RPS)])
    for t in range(3):
        off = base + t * CHUNK
        pltpu.sync_copy(src_hbm.at[pl.ds(off, CHUNK)], sidx[t].at[0])
        pltpu.sync_copy(dst_hbm.at[pl.ds(off, CHUNK)], didx[t].at[0])
    plsc.subcore_barrier()

    pltpu.async_copy(y_hbm.at[sidx[0].at[0]], rows[0], semg[0])
    pltpu.async_copy(y_hbm.at[sidx[1].at[0]], rows[1], semg[1])
    for j in range(CPW):
        b = j % 4
        q = j % 5
        if j + 2 < CPW:
            if j >= 1:
                pltpu.make_async_copy(src_hbm.at[pl.ds(base, CHUNK)],
                                      sidx[(j + 2) % 5].at[0],
                                      semi[(j + 2) % 5]).wait()
                pltpu.make_async_copy(dst_hbm.at[pl.ds(base, CHUNK)],
                                      didx[(j + 2) % 5].at[0],
                                      semi[(j + 2) % 5]).wait()
            if j >= 2:
                pltpu.make_async_copy(rows[(j - 2) % 4],
                                      acc.at[didx[(j - 2) % 5].at[0]],
                                      sems[(j - 2) % 4]).wait()
            pltpu.async_copy(y_hbm.at[sidx[(j + 2) % 5].at[0]],
                             rows[(j + 2) % 4], semg[(j + 2) % 4])
        pltpu.make_async_copy(y_hbm.at[sidx[q].at[0]], rows[b],
                              semg[b]).wait()
        pltpu.async_copy(rows[b], acc.at[didx[q].at[0]], sems[b], add=True)
        if j + 3 < CPW:
            off = base + (j + 3) * CHUNK
            pltpu.async_copy(src_hbm.at[pl.ds(off, CHUNK)],
                             sidx[(j + 3) % 5].at[0], semi[(j + 3) % 5])
            pltpu.async_copy(dst_hbm.at[pl.ds(off, CHUNK)],
                             didx[(j + 3) % 5].at[0], semi[(j + 3) % 5])
    for k in (CPW - 4, CPW - 3, CPW - 2, CPW - 1):
        pltpu.make_async_copy(rows[k % 4], acc.at[didx[k % 5].at[0]],
                              sems[k % 4]).wait()
    plsc.subcore_barrier()
    pltpu.sync_copy(acc.at[pl.ds(r0, RPS)], out_hbm.at[c, pl.ds(r0, RPS)])


def _sc_edges_call(y, src_p, dst_p):
    idx = pltpu.VMEM((1, CHUNK), jnp.int32)
    rbuf = pltpu.VMEM((CHUNK, D), jnp.float32)
    sem = pltpu.SemaphoreType.DMA
    return pl.kernel(
        _sc_edges_body,
        out_type=jax.ShapeDtypeStruct((NC, N_PAD, D), jnp.float32),
        mesh=_sc_mesh(),
        scratch_types=[idx] * 10 + [rbuf] * 4 + [
            pltpu.VMEM_SHARED((N_PAD, D), jnp.float32)] + [sem] * 13,
    )(y, src_p, dst_p)


def _matT(a, w):
    return lax.dot_general(a, w, (((1,), (1,)), ((), ())),
                           preferred_element_type=jnp.float32)


def _tc_prep_body(x_ref, w_ref, hist_ref, y_ref, dinv_ref):
    deg = 1.0 + jnp.sum(hist_ref[...], axis=0)
    dinv = lax.rsqrt(deg)[:, None]
    y_ref[...] = _matT(x_ref[...], w_ref[...]) * dinv
    dinv_ref[...] = dinv


def _tc_prep(x_p, w1, hist):
    return pl.pallas_call(
        _tc_prep_body,
        grid=(NBLK,),
        in_specs=[
            pl.BlockSpec((BLK, D), lambda i: (i, 0)),
            pl.BlockSpec((D, D), lambda i: (0, 0)),
            pl.BlockSpec((NC, BLK), lambda i: (0, i)),
        ],
        out_specs=[
            pl.BlockSpec((BLK, D), lambda i: (i, 0)),
            pl.BlockSpec((BLK, 1), lambda i: (i, 0)),
        ],
        out_shape=[
            jax.ShapeDtypeStruct((N_PAD, D), jnp.float32),
            jax.ShapeDtypeStruct((N_PAD, 1), jnp.float32),
        ],
    )(x_p, w1, hist)


def _tc_layer_body(p_ref, y_ref, dinv_ref, b_ref, gamma_ref, beta_ref, w_ref,
                   out_ref):
    p = p_ref[...]
    dinv = dinv_ref[...]
    g = dinv * (p[0] + p[1] - y_ref[...])
    h = (g + b_ref[...]) * (gamma_ref[...] * BN_SCALE) + beta_ref[...]
    h = jnp.maximum(h, 0.0)
    out_ref[...] = _matT(h, w_ref[...]) * dinv


def _tc_layer(p, y, dinv, b, gamma, beta, w_next):
    return pl.pallas_call(
        _tc_layer_body,
        grid=(NBLK,),
        in_specs=[
            pl.BlockSpec((NC, BLK, D), lambda i: (0, i, 0)),
            pl.BlockSpec((BLK, D), lambda i: (i, 0)),
            pl.BlockSpec((BLK, 1), lambda i: (i, 0)),
            pl.BlockSpec((1, D), lambda i: (0, 0)),
            pl.BlockSpec((1, D), lambda i: (0, 0)),
            pl.BlockSpec((1, D), lambda i: (0, 0)),
            pl.BlockSpec((D, D), lambda i: (0, 0)),
        ],
        out_specs=pl.BlockSpec((BLK, D), lambda i: (i, 0)),
        out_shape=jax.ShapeDtypeStruct((N_PAD, D), jnp.float32),
    )(p, y, dinv, b, gamma, beta, w_next)


def _tc_final_body(p_ref, y_ref, dinv_ref, b_ref, gamma_ref, beta_ref,
                   pw1_ref, pb1_ref, pw2_ref, pb2_ref, ne_ref, ge_ref,
                   acc_ref):
    i = pl.program_id(0)

    @pl.when(i == 0)
    def _():
        acc_ref[...] = jnp.zeros_like(acc_ref)

    p = p_ref[...]
    g = dinv_ref[...] * (p[0] + p[1] - y_ref[...])
    emb = (g + b_ref[...]) * (gamma_ref[...] * BN_SCALE) + beta_ref[...]
    ne_ref[...] = emb
    row = i * BLK + lax.broadcasted_iota(jnp.int32, (BLK, 1), 0)
    masked = jnp.where(row < N, emb, 0.0)
    acc_ref[...] += jnp.sum(masked, axis=0, keepdims=True)

    @pl.when(i == NBLK - 1)
    def _():
        gm = acc_ref[...] * (1.0 / N)
        g1 = jnp.maximum(_matT(gm, pw1_ref[...]) + pb1_ref[...], 0.0)
        ge_ref[...] = _matT(g1, pw2_ref[...]) + pb2_ref[...]


def _tc_final(p, y, dinv, b, gamma, beta, pw1, pb1, pw2, pb2):
    return pl.pallas_call(
        _tc_final_body,
        grid=(NBLK,),
        in_specs=[
            pl.BlockSpec((NC, BLK, D), lambda i: (0, i, 0)),
            pl.BlockSpec((BLK, D), lambda i: (i, 0)),
            pl.BlockSpec((BLK, 1), lambda i: (i, 0)),
            pl.BlockSpec((1, D), lambda i: (0, 0)),
            pl.BlockSpec((1, D), lambda i: (0, 0)),
            pl.BlockSpec((1, D), lambda i: (0, 0)),
            pl.BlockSpec((D, D), lambda i: (0, 0)),
            pl.BlockSpec((1, D), lambda i: (0, 0)),
            pl.BlockSpec((D, D), lambda i: (0, 0)),
            pl.BlockSpec((1, D), lambda i: (0, 0)),
        ],
        out_specs=[
            pl.BlockSpec((BLK, D), lambda i: (i, 0)),
            pl.BlockSpec((1, D), lambda i: (0, 0)),
        ],
        out_shape=[
            jax.ShapeDtypeStruct((N_PAD, D), jnp.float32),
            jax.ShapeDtypeStruct((1, D), jnp.float32),
        ],
        scratch_shapes=[pltpu.VMEM((1, D), jnp.float32)],
    )(p, y, dinv, b, gamma, beta, pw1, pb1, pw2, pb2)


def kernel(x, edge_index, W1, b1, gamma1, beta1, W2, b2, gamma2, beta2,
           W3, b3, gamma3, beta3, PW1, Pb1, PW2, Pb2):
    src = edge_index[0].astype(jnp.int32)
    dst = edge_index[1].astype(jnp.int32)
    pad = jnp.full((E_PAD - E,), TRASH, dtype=jnp.int32)
    src_p = jnp.concatenate([src, pad])
    dst_p = jnp.concatenate([dst, pad])
    x_p = jnp.concatenate(
        [x, jnp.zeros((N_PAD - N, D), dtype=jnp.float32)], axis=0)
    ones_c = jnp.ones((CHUNK,), dtype=jnp.float32)
    zeros_rows = jnp.zeros((N_PAD,), dtype=jnp.float32)

    b1r, g1r, be1 = b1[None, :], gamma1[None, :], beta1[None, :]
    b2r, g2r, be2 = b2[None, :], gamma2[None, :], beta2[None, :]
    b3r, g3r, be3 = b3[None, :], gamma3[None, :], beta3[None, :]

    hist = _sc_deg_call(dst_p, ones_c, zeros_rows).reshape(NC, N_PAD)
    y1, dinv = _tc_prep(x_p, W1, hist)
    p1_ = _sc_edges_call(y1, src_p, dst_p)
    y2 = _tc_layer(p1_, y1, dinv, b1r, g1r, be1, W2)
    p2_ = _sc_edges_call(y2, src_p, dst_p)
    y3 = _tc_layer(p2_, y2, dinv, b2r, g2r, be2, W3)
    p3_ = _sc_edges_call(y3, src_p, dst_p)
    ne_pad, ge = _tc_final(p3_, y3, dinv, b3r, g3r, be3, PW1, Pb1[None, :],
                           PW2, Pb2[None, :])
    return ne_pad[:N], ge
